# baseline (speedup 1.0000x reference)
import functools

import jax
import jax.numpy as jnp
from jax import lax
from jax.experimental import pallas as pl
from jax.experimental.pallas import tpu as pltpu
from jax.experimental.pallas import tpu_sc as plsc


def _sc_gather(idx, table):
    B = idx.shape[0]
    _, D = table.shape
    info = plsc.get_sparse_core_info()
    nw = info.num_cores * info.num_subcores
    b_per_w = B // nw
    mesh = plsc.VectorSubcoreMesh(core_axis_name="c", subcore_axis_name="s")

    @functools.partial(
        pl.kernel,
        mesh=mesh,
        out_type=jax.ShapeDtypeStruct((B, D), jnp.float32),
        scratch_types=[
            pltpu.VMEM((b_per_w,), jnp.int32),
            pltpu.VMEM((b_per_w, D), jnp.float32),
            pltpu.SemaphoreType.DMA,
        ],
        compiler_params=pltpu.CompilerParams(use_tc_tiling_on_sc=False),
    )
    def gather_kernel(idx_hbm, table_hbm, out_hbm, idx_v, rows_v, sem):
        wid = lax.axis_index("s") * info.num_cores + lax.axis_index("c")
        base = wid * b_per_w
        pltpu.sync_copy(idx_hbm.at[pl.ds(base, b_per_w)], idx_v)
        pltpu.async_copy(table_hbm.at[idx_v], rows_v, sem).wait()
        pltpu.sync_copy(rows_v, out_hbm.at[pl.ds(base, b_per_w)])

    return gather_kernel(idx, table)


_TILE_V = 2048


def _tc_matmul(e, w):
    B, D = e.shape
    V = w.shape[0]

    def body(e_ref, w_ref, o_ref):
        o_ref[...] = lax.dot_general(
            e_ref[...], w_ref[...],
            (((1,), (1,)), ((), ())),
            preferred_element_type=jnp.float32,
        )

    return pl.pallas_call(
        body,
        grid=(pl.cdiv(V, _TILE_V),),
        in_specs=[
            pl.BlockSpec((B, D), lambda i: (0, 0)),
            pl.BlockSpec((_TILE_V, D), lambda i: (i, 0)),
        ],
        out_specs=pl.BlockSpec((B, _TILE_V), lambda i: (0, i)),
        out_shape=jax.ShapeDtypeStruct((B, V), jnp.float32),
        compiler_params=pltpu.CompilerParams(
            dimension_semantics=("arbitrary",),
        ),
    )(e, w)


def kernel(x, embed_weight, linear_weight):
    e = _sc_gather(x.astype(jnp.int32), embed_weight)
    return _tc_matmul(e, linear_weight)


# speedup vs baseline: 2.8256x; 2.8256x over previous
import functools

import jax
import jax.numpy as jnp
from jax import lax
from jax.experimental import pallas as pl
from jax.experimental.pallas import tpu as pltpu
from jax.experimental.pallas import tpu_sc as plsc


def _sc_gather(idx, table):
    B = idx.shape[0]
    _, D = table.shape
    info = plsc.get_sparse_core_info()
    nw = info.num_cores * info.num_subcores
    b_per_w = B // nw
    mesh = plsc.VectorSubcoreMesh(core_axis_name="c", subcore_axis_name="s")

    @functools.partial(
        pl.kernel,
        mesh=mesh,
        out_type=jax.ShapeDtypeStruct((B, D), jnp.float32),
        scratch_types=[
            pltpu.VMEM((b_per_w,), jnp.int32),
            pltpu.VMEM((b_per_w, D), jnp.float32),
            pltpu.SemaphoreType.DMA,
        ],
        compiler_params=pltpu.CompilerParams(use_tc_tiling_on_sc=False),
    )
    def gather_kernel(idx_hbm, table_hbm, out_hbm, idx_v, rows_v, sem):
        wid = lax.axis_index("s") * info.num_cores + lax.axis_index("c")
        base = wid * b_per_w
        pltpu.sync_copy(idx_hbm.at[pl.ds(base, b_per_w)], idx_v)
        pltpu.async_copy(table_hbm.at[idx_v], rows_v, sem).wait()
        pltpu.sync_copy(rows_v, out_hbm.at[pl.ds(base, b_per_w)])

    return gather_kernel(idx, table)


_TILE_V = 2048


def _tc_matmul_t(wt, e):
    D, V = wt.shape
    B = e.shape[0]

    def body(wt_ref, e_ref, o_ref):
        o_ref[...] = lax.dot_general(
            wt_ref[...], e_ref[...],
            (((0,), (1,)), ((), ())),
            preferred_element_type=jnp.float32,
        )

    return pl.pallas_call(
        body,
        grid=(pl.cdiv(V, _TILE_V),),
        in_specs=[
            pl.BlockSpec((D, _TILE_V), lambda i: (0, i)),
            pl.BlockSpec((B, D), lambda i: (0, 0)),
        ],
        out_specs=pl.BlockSpec((_TILE_V, B), lambda i: (i, 0)),
        out_shape=jax.ShapeDtypeStruct((V, B), jnp.float32),
        compiler_params=pltpu.CompilerParams(
            dimension_semantics=("arbitrary",),
        ),
    )(wt, e)


def kernel(x, embed_weight, linear_weight):
    e = _sc_gather(x.astype(jnp.int32), embed_weight)
    out_t = _tc_matmul_t(linear_weight.T, e)
    return out_t.T


# speedup vs baseline: 3.6904x; 1.3061x over previous
import functools

import jax
import jax.numpy as jnp
from jax import lax
from jax.experimental import pallas as pl
from jax.experimental.pallas import tpu as pltpu
from jax.experimental.pallas import tpu_sc as plsc


def _sc_gather_t(idx, emb_t):
    D, V = emb_t.shape
    B = idx.shape[0]
    info = plsc.get_sparse_core_info()
    nw = info.num_cores * info.num_subcores
    d_per_w = D // nw
    mesh = plsc.VectorSubcoreMesh(core_axis_name="c", subcore_axis_name="s")

    @functools.partial(
        pl.kernel,
        mesh=mesh,
        out_type=jax.ShapeDtypeStruct((D, B), jnp.float32),
        scratch_types=[
            pltpu.VMEM((B,), jnp.int32),
            pltpu.VMEM((V,), jnp.float32),
            pltpu.VMEM((B,), jnp.float32),
        ],
        compiler_params=pltpu.CompilerParams(needs_layout_passes=False),
    )
    def gather_kernel(idx_hbm, emb_hbm, out_hbm, idx_v, row_v, ot_v):
        wid = lax.axis_index("s") * info.num_cores + lax.axis_index("c")
        pltpu.sync_copy(idx_hbm, idx_v)
        for r in range(d_per_w):
            d = wid * d_per_w + r
            pltpu.sync_copy(emb_hbm.at[d], row_v)
            for j in range(B // 16):
                ids = idx_v[pl.ds(j * 16, 16)]
                ot_v[pl.ds(j * 16, 16)] = plsc.load_gather(row_v, [ids])
            pltpu.sync_copy(ot_v, out_hbm.at[d])

    return gather_kernel(idx, emb_t)


_TILE_V = 2048


def _tc_matmul_t(wt, et):
    D, V = wt.shape
    B = et.shape[1]

    def body(wt_ref, et_ref, o_ref):
        o_ref[...] = lax.dot_general(
            wt_ref[...], et_ref[...],
            (((0,), (0,)), ((), ())),
            preferred_element_type=jnp.float32,
        )

    return pl.pallas_call(
        body,
        grid=(pl.cdiv(V, _TILE_V),),
        in_specs=[
            pl.BlockSpec((D, _TILE_V), lambda i: (0, i)),
            pl.BlockSpec((D, B), lambda i: (0, 0)),
        ],
        out_specs=pl.BlockSpec((_TILE_V, B), lambda i: (i, 0)),
        out_shape=jax.ShapeDtypeStruct((V, B), jnp.float32),
        compiler_params=pltpu.CompilerParams(
            dimension_semantics=("arbitrary",),
        ),
    )(wt, et)


def kernel(x, embed_weight, linear_weight):
    et = _sc_gather_t(x.astype(jnp.int32), embed_weight.T)
    out_t = _tc_matmul_t(linear_weight.T, et)
    return out_t.T


# speedup vs baseline: 3.7422x; 1.0140x over previous
import functools

import jax
import jax.numpy as jnp
from jax import lax
from jax.experimental import pallas as pl
from jax.experimental.pallas import tpu as pltpu
from jax.experimental.pallas import tpu_sc as plsc


def _sc_gather_t(idx, emb_t):
    D, V = emb_t.shape
    B = idx.shape[0]
    info = plsc.get_sparse_core_info()
    nw = info.num_cores * info.num_subcores
    d_per_w = D // nw
    mesh = plsc.VectorSubcoreMesh(core_axis_name="c", subcore_axis_name="s")

    @functools.partial(
        pl.kernel,
        mesh=mesh,
        out_type=jax.ShapeDtypeStruct((D, B), jnp.float32),
        scratch_types=[
            pltpu.VMEM((B,), jnp.int32),
            pltpu.VMEM((V,), jnp.float32),
            pltpu.VMEM((B,), jnp.float32),
        ],
        compiler_params=pltpu.CompilerParams(needs_layout_passes=False),
    )
    def gather_kernel(idx_hbm, emb_hbm, out_hbm, idx_v, row_v, ot_v):
        wid = lax.axis_index("s") * info.num_cores + lax.axis_index("c")
        pltpu.sync_copy(idx_hbm, idx_v)
        for r in range(d_per_w):
            d = wid * d_per_w + r
            pltpu.sync_copy(emb_hbm.at[d], row_v)
            for j in range(B // 16):
                ids = idx_v[pl.ds(j * 16, 16)]
                ot_v[pl.ds(j * 16, 16)] = plsc.load_gather(row_v, [ids])
            pltpu.sync_copy(ot_v, out_hbm.at[d])

    return gather_kernel(idx, emb_t)


_TILE_V = 4096


def _tc_matmul_t(wt, et):
    D, V = wt.shape
    B = et.shape[1]

    def body(wt_ref, et_ref, o_ref):
        o_ref[...] = lax.dot_general(
            wt_ref[...], et_ref[...],
            (((0,), (0,)), ((), ())),
            preferred_element_type=jnp.float32,
        )

    return pl.pallas_call(
        body,
        grid=(pl.cdiv(V, _TILE_V),),
        in_specs=[
            pl.BlockSpec((D, _TILE_V), lambda i: (0, i)),
            pl.BlockSpec((D, B), lambda i: (0, 0)),
        ],
        out_specs=pl.BlockSpec((_TILE_V, B), lambda i: (i, 0)),
        out_shape=jax.ShapeDtypeStruct((V, B), jnp.float32),
        compiler_params=pltpu.CompilerParams(
            dimension_semantics=("arbitrary",),
        ),
    )(wt, et)


def kernel(x, embed_weight, linear_weight):
    et = _sc_gather_t(x.astype(jnp.int32), embed_weight.T)
    out_t = _tc_matmul_t(linear_weight.T, et)
    return out_t.T


# speedup vs baseline: 3.7601x; 1.0048x over previous
import functools

import jax
import jax.numpy as jnp
from jax import lax
from jax.experimental import pallas as pl
from jax.experimental.pallas import tpu as pltpu
from jax.experimental.pallas import tpu_sc as plsc


def _sc_gather_t(idx, emb_t):
    D, V = emb_t.shape
    B = idx.shape[0]
    info = plsc.get_sparse_core_info()
    nw = info.num_cores * info.num_subcores
    d_per_w = D // nw
    mesh = plsc.VectorSubcoreMesh(core_axis_name="c", subcore_axis_name="s")

    @functools.partial(
        pl.kernel,
        mesh=mesh,
        out_type=jax.ShapeDtypeStruct((D, B), jnp.float32),
        scratch_types=[
            pltpu.VMEM((B,), jnp.int32),
            pltpu.VMEM((V,), jnp.float32),
            pltpu.VMEM((B,), jnp.float32),
        ],
        compiler_params=pltpu.CompilerParams(needs_layout_passes=False),
    )
    def gather_kernel(idx_hbm, emb_hbm, out_hbm, idx_v, row_v, ot_v):
        wid = lax.axis_index("s") * info.num_cores + lax.axis_index("c")
        pltpu.sync_copy(idx_hbm, idx_v)
        for r in range(d_per_w):
            d = wid * d_per_w + r
            pltpu.sync_copy(emb_hbm.at[d], row_v)
            for j in range(B // 16):
                ids = idx_v[pl.ds(j * 16, 16)]
                ot_v[pl.ds(j * 16, 16)] = plsc.load_gather(row_v, [ids])
            pltpu.sync_copy(ot_v, out_hbm.at[d])

    return gather_kernel(idx, emb_t)


_TILE_V = 6144


def _tc_matmul_t(wt, et):
    D, V = wt.shape
    B = et.shape[1]

    def body(wt_ref, et_ref, o_ref):
        o_ref[...] = lax.dot_general(
            wt_ref[...], et_ref[...],
            (((0,), (0,)), ((), ())),
            preferred_element_type=jnp.float32,
        )

    return pl.pallas_call(
        body,
        grid=(pl.cdiv(V, _TILE_V),),
        in_specs=[
            pl.BlockSpec((D, _TILE_V), lambda i: (0, i)),
            pl.BlockSpec((D, B), lambda i: (0, 0)),
        ],
        out_specs=pl.BlockSpec((_TILE_V, B), lambda i: (i, 0)),
        out_shape=jax.ShapeDtypeStruct((V, B), jnp.float32),
        compiler_params=pltpu.CompilerParams(
            dimension_semantics=("arbitrary",),
        ),
    )(wt, et)


def kernel(x, embed_weight, linear_weight):
    et = _sc_gather_t(x.astype(jnp.int32), embed_weight.T)
    out_t = _tc_matmul_t(linear_weight.T, et)
    return out_t.T
